# baseline (speedup 1.0000x reference)
import jax
import jax.numpy as jnp
from jax import lax
from jax.experimental import pallas as pl
from jax.experimental.pallas import tpu as pltpu
from jax.experimental.pallas import tpu_sc as plsc

_NC = 2
_NS = 16
_G = 64


def _pick_chunk(ept):
    for ch in range(128, 0, -1):
        if ept % ch == 0 and (ept // ch) % 2 == 0:
            return ch
    raise ValueError(f"no legal chunk size for {ept} edges per tile")


def _edge_pass(feats, src, dst, with_count):
    n, h = feats.shape
    e = src.shape[0]
    nw = _NC * _NS
    assert e % nw == 0 and n % _NS == 0
    ept = e // nw
    ch = _pick_chunk(ept)
    nch = ept // ch
    rp = (n // 8 // _NS) * 8
    rl = n - rp * (_NS - 1)

    mesh = plsc.VectorSubcoreMesh(core_axis_name="c", subcore_axis_name="s",
                                  num_cores=_NC, num_subcores=_NS)

    assert nch % 2 == 0

    out_type = [jax.ShapeDtypeStruct((_NC * n, h), jnp.float32)]
    scratch = [
        pltpu.VMEM_SHARED((n, h), jnp.float32),
        pltpu.VMEM((nch, ch), jnp.int32),
        pltpu.VMEM((nch, ch), jnp.int32),
        pltpu.VMEM((2, ch, h), jnp.float32),
        pltpu.VMEM((rl, h), jnp.float32),
        pltpu.SemaphoreType.DMA,
        pltpu.SemaphoreType.DMA,
        pltpu.SemaphoreType.DMA,
        pltpu.SemaphoreType.DMA,
    ]
    if with_count:
        out_type.append(jax.ShapeDtypeStruct((_NC * n,), jnp.float32))
        scratch += [
            pltpu.VMEM_SHARED((n,), jnp.float32),
            pltpu.VMEM((ch,), jnp.float32),
            pltpu.VMEM((rl,), jnp.float32),
            pltpu.SemaphoreType.DMA,
            pltpu.SemaphoreType.DMA,
        ]

    def body(feats_hbm, zrows_hbm, zcnt_hbm, ones_hbm, src_hbm, dst_hbm,
             *rest):
        if with_count:
            (agg_out, cnt_out, agg_sh, src_v, dst_v, rows_v, zb_v,
             gs0, gs1, ss0, ss1, cnt_sh, ones_v, zc_v, cs0, cs1) = rest
        else:
            (agg_out, agg_sh, src_v, dst_v, rows_v, zb_v,
             gs0, gs1, ss0, ss1) = rest
            cs0 = cs1 = None
        gsems = (gs0, gs1)
        ssems = (ss0, ss1)
        csems = (cs0, cs1)
        c = lax.axis_index("c")
        s = lax.axis_index("s")
        wid = c * _NS + s

        pltpu.sync_copy(src_hbm.at[pl.ds(wid * nch, nch)], src_v)
        pltpu.sync_copy(dst_hbm.at[pl.ds(wid * nch, nch)], dst_v)

        pltpu.sync_copy(zrows_hbm, zb_v)
        if with_count:
            pltpu.sync_copy(zcnt_hbm, zc_v)
            pltpu.sync_copy(ones_hbm, ones_v)

        @pl.when(s < _NS - 1)
        def _():
            pltpu.sync_copy(zb_v.at[pl.ds(0, rp)],
                            agg_sh.at[pl.ds(s * rp, rp)])
            if with_count:
                pltpu.sync_copy(zc_v.at[pl.ds(0, rp)],
                                cnt_sh.at[pl.ds(s * rp, rp)])

        @pl.when(s == _NS - 1)
        def _():
            pltpu.sync_copy(zb_v, agg_sh.at[pl.ds((_NS - 1) * rp, rl)])
            if with_count:
                pltpu.sync_copy(zc_v, cnt_sh.at[pl.ds((_NS - 1) * rp, rl)])

        plsc.subcore_barrier()

        def fire_gather(j, b):
            pltpu.async_copy(feats_hbm.at[src_v.at[j]],
                             rows_v.at[b], gsems[b])

        def wait_gather(j, b):
            pltpu.make_async_copy(feats_hbm.at[src_v.at[j]],
                                  rows_v.at[b], gsems[b]).wait()

        def fire_scatter(j, b):
            pltpu.async_copy(rows_v.at[b], agg_sh.at[dst_v.at[j]],
                             ssems[b], add=True)
            if with_count:
                pltpu.async_copy(ones_v, cnt_sh.at[dst_v.at[j]],
                                 csems[b], add=True)

        def wait_scatter(j, b):
            pltpu.make_async_copy(rows_v.at[b], agg_sh.at[dst_v.at[j]],
                                  ssems[b]).wait()
            if with_count:
                pltpu.make_async_copy(ones_v, cnt_sh.at[dst_v.at[j]],
                                      csems[b]).wait()

        fire_gather(0, 0)

        def pair(p, carry):
            j = 2 * p
            wait_gather(j, 0)
            fire_scatter(j, 0)

            @pl.when(p > 0)
            def _():
                wait_scatter(j - 1, 1)

            fire_gather(j + 1, 1)
            wait_gather(j + 1, 1)
            fire_scatter(j + 1, 1)
            wait_scatter(j, 0)

            @pl.when(p < nch // 2 - 1)
            def _():
                fire_gather(j + 2, 0)

            return carry

        lax.fori_loop(0, nch // 2, pair, 0)
        wait_scatter(nch - 1, 1)
        plsc.subcore_barrier()

        @pl.when(s < _NS - 1)
        def _():
            pltpu.sync_copy(agg_sh.at[pl.ds(s * rp, rp)],
                            zb_v.at[pl.ds(0, rp)])
            pltpu.sync_copy(zb_v.at[pl.ds(0, rp)],
                            agg_out.at[pl.ds(c * n + s * rp, rp)])
            if with_count:
                pltpu.sync_copy(cnt_sh.at[pl.ds(s * rp, rp)],
                                zc_v.at[pl.ds(0, rp)])
                pltpu.sync_copy(zc_v.at[pl.ds(0, rp)],
                                cnt_out.at[pl.ds(c * n + s * rp, rp)])

        @pl.when(s == _NS - 1)
        def _():
            pltpu.sync_copy(agg_sh.at[pl.ds((_NS - 1) * rp, rl)], zb_v)
            pltpu.sync_copy(zb_v,
                            agg_out.at[pl.ds(c * n + (_NS - 1) * rp, rl)])
            if with_count:
                pltpu.sync_copy(cnt_sh.at[pl.ds((_NS - 1) * rp, rl)], zc_v)
                pltpu.sync_copy(
                    zc_v, cnt_out.at[pl.ds(c * n + (_NS - 1) * rp, rl)])

    run = pl.kernel(
        body, out_type=tuple(out_type), mesh=mesh,
        scratch_types=tuple(scratch),
        compiler_params=pltpu.CompilerParams(use_tc_tiling_on_sc=False))
    zrows = jnp.zeros((rl, h), jnp.float32)
    zcnt = jnp.zeros((rl,), jnp.float32)
    ones = jnp.ones((ch,), jnp.float32)
    return run(feats, zrows, zcnt, ones,
               src.reshape(-1, ch), dst.reshape(-1, ch))


def _proj_tc(x, wl1t, wr1t):
    n = x.shape[0]
    h = wl1t.shape[1]

    def body(x_ref, wl_ref, wr_ref, xl_ref, xr_ref):
        xv = x_ref[...]
        xl_ref[...] = jnp.dot(xv, wl_ref[...],
                              preferred_element_type=jnp.float32)
        xr_ref[...] = jnp.dot(xv, wr_ref[...],
                              preferred_element_type=jnp.float32)

    return pl.pallas_call(
        body,
        out_shape=(jax.ShapeDtypeStruct((n, h), jnp.float32),
                   jax.ShapeDtypeStruct((n, h), jnp.float32)),
    )(x, wl1t, wr1t)


def _mid_tc(agg1p, cnt2, xr, bl1, wl2t, wr2t):
    n, h = xr.shape

    def body(agg_ref, cnt_ref, xr_ref, b_ref, wl_ref, wr_ref,
             hl_ref, hr_ref, sc_ref):
        a = agg_ref[...]
        agg = a[:n] + a[n:]
        cv = cnt_ref[...]
        cnt = jnp.maximum(cv[:n] + cv[n:], 1.0)
        inv = 1.0 / cnt
        hh = jnp.maximum(agg * inv + b_ref[...] + xr_ref[...], 0.0)
        hl_ref[...] = jnp.dot(hh, wl_ref[...],
                              preferred_element_type=jnp.float32)
        hr_ref[...] = jnp.dot(hh, wr_ref[...],
                              preferred_element_type=jnp.float32)
        sc_ref[...] = inv

    return pl.pallas_call(
        body,
        out_shape=(jax.ShapeDtypeStruct((n, h), jnp.float32),
                   jax.ShapeDtypeStruct((n, h), jnp.float32),
                   jax.ShapeDtypeStruct((n, 1), jnp.float32)),
    )(agg1p, cnt2, xr, bl1.reshape(1, h), wl2t, wr2t)


def _final_tc(agg2p, scale, hr, bl2, batch_row, wfct, bfc):
    n, h = hr.shape
    co = wfct.shape[1]

    def body(agg_ref, sc_ref, hr_ref, b_ref, bt_ref, wf_ref, bf_ref, o_ref):
        a = agg_ref[...]
        h2 = (a[:n] + a[n:]) * sc_ref[...] + b_ref[...] + hr_ref[...]
        ids = bt_ref[...]
        iot = lax.broadcasted_iota(jnp.int32, (_G, n), 0)
        oh = jnp.where(iot == ids, 1.0, 0.0)
        pooled = jnp.dot(oh, h2, preferred_element_type=jnp.float32)
        gcnt = jnp.sum(oh, axis=1, keepdims=True)
        pooled = pooled / jnp.maximum(gcnt, 1.0)
        logits = jnp.dot(pooled, wf_ref[...],
                         preferred_element_type=jnp.float32) + bf_ref[...]
        m = jnp.max(logits, axis=1, keepdims=True)
        sh = logits - m
        o_ref[...] = sh - jnp.log(jnp.sum(jnp.exp(sh), axis=1, keepdims=True))

    return pl.pallas_call(
        body,
        out_shape=jax.ShapeDtypeStruct((_G, co), jnp.float32),
    )(agg2p, scale, hr, bl2.reshape(1, h), batch_row, wfct,
      bfc.reshape(1, co))


def kernel(x, edge_index, batch, Wl1, bl1, Wr1, Wl2, bl2, Wr2, Wfc, bfc):
    n, _ = x.shape
    h = Wl1.shape[0]
    src = edge_index[0]
    dst = edge_index[1]

    xl, xr = _proj_tc(x, Wl1.T, Wr1.T)
    agg1p, cntp = _edge_pass(xl, src, dst, with_count=True)
    hl, hr, scale = _mid_tc(agg1p, cntp.reshape(-1, 1), xr, bl1,
                            Wl2.T, Wr2.T)
    (agg2p,) = _edge_pass(hl, src, dst, with_count=False)
    return _final_tc(agg2p, scale, hr, bl2, batch.reshape(1, n), Wfc.T, bfc)


# speedup vs baseline: 1.4317x; 1.4317x over previous
import jax
import jax.numpy as jnp
from jax import lax
from jax.experimental import pallas as pl
from jax.experimental.pallas import tpu as pltpu
from jax.experimental.pallas import tpu_sc as plsc

_NC = 2
_NS = 16
_G = 64


def _pick_chunk(ept):
    for ch in range(128, 0, -1):
        nch = ept // ch
        if ept % ch == 0 and nch % 4 == 0 and nch >= 8:
            return ch
    raise ValueError(f"no legal chunk size for {ept} edges per tile")


def _edge_pass(feats, src, dst, with_count):
    n, h = feats.shape
    e = src.shape[0]
    nw = _NC * _NS
    assert e % nw == 0 and n % _NS == 0
    ept = e // nw
    ch = _pick_chunk(ept)
    nch = ept // ch
    rp = (n // 8 // _NS) * 8
    rl = n - rp * (_NS - 1)

    mesh = plsc.VectorSubcoreMesh(core_axis_name="c", subcore_axis_name="s",
                                  num_cores=_NC, num_subcores=_NS)

    assert nch % 4 == 0 and nch >= 8
    _NB = 4

    out_type = [jax.ShapeDtypeStruct((_NC * n, h), jnp.float32)]
    scratch = [
        pltpu.VMEM_SHARED((n, h), jnp.float32),
        pltpu.VMEM((nch, ch), jnp.int32),
        pltpu.VMEM((nch, ch), jnp.int32),
        pltpu.VMEM((_NB, ch, h), jnp.float32),
        pltpu.VMEM((rl, h), jnp.float32),
    ]
    scratch += [pltpu.SemaphoreType.DMA] * _NB
    scratch += [pltpu.SemaphoreType.DMA] * _NB
    if with_count:
        out_type.append(jax.ShapeDtypeStruct((_NC * n,), jnp.float32))
        scratch += [
            pltpu.VMEM_SHARED((n,), jnp.float32),
            pltpu.VMEM((ch,), jnp.float32),
            pltpu.VMEM((rl,), jnp.float32),
        ]
        scratch += [pltpu.SemaphoreType.DMA] * _NB

    def body(feats_hbm, zrows_hbm, zcnt_hbm, ones_hbm, edges_hbm, *rest):
        if with_count:
            (agg_out, cnt_out, agg_sh, src_v, dst_v, rows_v, zb_v,
             *sems) = rest
            gsems = sems[:_NB]
            ssems = sems[_NB:2 * _NB]
            cnt_sh, ones_v, zc_v = sems[2 * _NB:2 * _NB + 3]
            csems = sems[2 * _NB + 3:]
        else:
            (agg_out, agg_sh, src_v, dst_v, rows_v, zb_v, *sems) = rest
            gsems = sems[:_NB]
            ssems = sems[_NB:]
            csems = (None,) * _NB
        c = lax.axis_index("c")
        s = lax.axis_index("s")
        wid = c * _NS + s

        pltpu.sync_copy(edges_hbm.at[0, pl.ds(wid * nch, nch)], src_v)
        pltpu.sync_copy(edges_hbm.at[1, pl.ds(wid * nch, nch)], dst_v)

        pltpu.sync_copy(zrows_hbm, zb_v)
        if with_count:
            pltpu.sync_copy(zcnt_hbm, zc_v)
            pltpu.sync_copy(ones_hbm, ones_v)

        @pl.when(s < _NS - 1)
        def _():
            pltpu.sync_copy(zb_v.at[pl.ds(0, rp)],
                            agg_sh.at[pl.ds(s * rp, rp)])
            if with_count:
                pltpu.sync_copy(zc_v.at[pl.ds(0, rp)],
                                cnt_sh.at[pl.ds(s * rp, rp)])

        @pl.when(s == _NS - 1)
        def _():
            pltpu.sync_copy(zb_v, agg_sh.at[pl.ds((_NS - 1) * rp, rl)])
            if with_count:
                pltpu.sync_copy(zc_v, cnt_sh.at[pl.ds((_NS - 1) * rp, rl)])

        plsc.subcore_barrier()

        def fire_gather(j, b):
            pltpu.async_copy(feats_hbm.at[src_v.at[j]],
                             rows_v.at[b], gsems[b])

        def wait_gather(j, b):
            pltpu.make_async_copy(feats_hbm.at[src_v.at[j]],
                                  rows_v.at[b], gsems[b]).wait()

        def fire_scatter(j, b):
            pltpu.async_copy(rows_v.at[b], agg_sh.at[dst_v.at[j]],
                             ssems[b], add=True)
            if with_count:
                pltpu.async_copy(ones_v, cnt_sh.at[dst_v.at[j]],
                                 csems[b], add=True)

        def wait_scatter(j, b):
            pltpu.make_async_copy(rows_v.at[b], agg_sh.at[dst_v.at[j]],
                                  ssems[b]).wait()
            if with_count:
                pltpu.make_async_copy(ones_v, cnt_sh.at[dst_v.at[j]],
                                      csems[b]).wait()

        def step(j, u, fire_next=True, wait_prev=True):
            wait_gather(j, u)
            fire_scatter(j, u)
            if wait_prev:
                wait_scatter(j - 1, (u + 3) % _NB)
            if fire_next:
                fire_gather(j + 3, (u + 3) % _NB)

        for b in range(_NB - 1):
            fire_gather(b, b)
        step(0, 0, wait_prev=False)
        for j in range(1, _NB):
            step(j, j % _NB)

        def quad(g, carry):
            j0 = _NB * g
            for u in range(_NB):
                j = j0 + u

                @pl.when(j + 3 < nch)
                def _():
                    step(j, u)

                @pl.when(j + 3 >= nch)
                def _():
                    step(j, u, fire_next=False)
            return carry

        lax.fori_loop(1, nch // _NB, quad, 0)
        wait_scatter(nch - 1, (_NB - 1) % _NB)
        plsc.subcore_barrier()

        @pl.when(s < _NS - 1)
        def _():
            pltpu.sync_copy(agg_sh.at[pl.ds(s * rp, rp)],
                            zb_v.at[pl.ds(0, rp)])
            pltpu.sync_copy(zb_v.at[pl.ds(0, rp)],
                            agg_out.at[pl.ds(c * n + s * rp, rp)])
            if with_count:
                pltpu.sync_copy(cnt_sh.at[pl.ds(s * rp, rp)],
                                zc_v.at[pl.ds(0, rp)])
                pltpu.sync_copy(zc_v.at[pl.ds(0, rp)],
                                cnt_out.at[pl.ds(c * n + s * rp, rp)])

        @pl.when(s == _NS - 1)
        def _():
            pltpu.sync_copy(agg_sh.at[pl.ds((_NS - 1) * rp, rl)], zb_v)
            pltpu.sync_copy(zb_v,
                            agg_out.at[pl.ds(c * n + (_NS - 1) * rp, rl)])
            if with_count:
                pltpu.sync_copy(cnt_sh.at[pl.ds((_NS - 1) * rp, rl)], zc_v)
                pltpu.sync_copy(
                    zc_v, cnt_out.at[pl.ds(c * n + (_NS - 1) * rp, rl)])

    run = pl.kernel(
        body, out_type=tuple(out_type), mesh=mesh,
        scratch_types=tuple(scratch),
        compiler_params=pltpu.CompilerParams(use_tc_tiling_on_sc=False))
    zrows = jnp.zeros((rl, h), jnp.float32)
    zcnt = jnp.zeros((rl,), jnp.float32)
    ones = jnp.ones((ch,), jnp.float32)
    edges = jnp.stack([src.reshape(-1, ch), dst.reshape(-1, ch)])
    return run(feats, zrows, zcnt, ones, edges)


def _proj_tc(x, wl1t, wr1t):
    n = x.shape[0]
    h = wl1t.shape[1]

    def body(x_ref, wl_ref, wr_ref, xl_ref, xr_ref):
        xv = x_ref[...]
        xl_ref[...] = jnp.dot(xv, wl_ref[...],
                              preferred_element_type=jnp.float32)
        xr_ref[...] = jnp.dot(xv, wr_ref[...],
                              preferred_element_type=jnp.float32)

    return pl.pallas_call(
        body,
        out_shape=(jax.ShapeDtypeStruct((n, h), jnp.float32),
                   jax.ShapeDtypeStruct((n, h), jnp.float32)),
    )(x, wl1t, wr1t)


def _mid_tc(agg1p, cnt2, xr, bl1, wl2t, wr2t):
    n, h = xr.shape

    def body(agg_ref, cnt_ref, xr_ref, b_ref, wl_ref, wr_ref,
             hl_ref, hr_ref, sc_ref):
        a = agg_ref[...]
        agg = a[:n] + a[n:]
        cv = cnt_ref[...]
        cnt = jnp.maximum(cv[:n] + cv[n:], 1.0)
        inv = 1.0 / cnt
        hh = jnp.maximum(agg * inv + b_ref[...] + xr_ref[...], 0.0)
        hl_ref[...] = jnp.dot(hh, wl_ref[...],
                              preferred_element_type=jnp.float32)
        hr_ref[...] = jnp.dot(hh, wr_ref[...],
                              preferred_element_type=jnp.float32)
        sc_ref[...] = inv

    return pl.pallas_call(
        body,
        out_shape=(jax.ShapeDtypeStruct((n, h), jnp.float32),
                   jax.ShapeDtypeStruct((n, h), jnp.float32),
                   jax.ShapeDtypeStruct((n, 1), jnp.float32)),
    )(agg1p, cnt2, xr, bl1.reshape(1, h), wl2t, wr2t)


def _final_tc(agg2p, scale, hr, bl2, batch_row, wfct, bfc):
    n, h = hr.shape
    co = wfct.shape[1]

    def body(agg_ref, sc_ref, hr_ref, b_ref, bt_ref, wf_ref, bf_ref, o_ref):
        a = agg_ref[...]
        h2 = (a[:n] + a[n:]) * sc_ref[...] + b_ref[...] + hr_ref[...]
        ids = bt_ref[...]
        iot = lax.broadcasted_iota(jnp.int32, (_G, n), 0)
        oh = jnp.where(iot == ids, 1.0, 0.0)
        pooled = jnp.dot(oh, h2, preferred_element_type=jnp.float32)
        gcnt = jnp.sum(oh, axis=1, keepdims=True)
        pooled = pooled / jnp.maximum(gcnt, 1.0)
        logits = jnp.dot(pooled, wf_ref[...],
                         preferred_element_type=jnp.float32) + bf_ref[...]
        m = jnp.max(logits, axis=1, keepdims=True)
        sh = logits - m
        o_ref[...] = sh - jnp.log(jnp.sum(jnp.exp(sh), axis=1, keepdims=True))

    return pl.pallas_call(
        body,
        out_shape=jax.ShapeDtypeStruct((_G, co), jnp.float32),
    )(agg2p, scale, hr, bl2.reshape(1, h), batch_row, wfct,
      bfc.reshape(1, co))


def kernel(x, edge_index, batch, Wl1, bl1, Wr1, Wl2, bl2, Wr2, Wfc, bfc):
    n, _ = x.shape
    h = Wl1.shape[0]
    src = edge_index[0]
    dst = edge_index[1]

    xl, xr = _proj_tc(x, Wl1.T, Wr1.T)
    agg1p, cntp = _edge_pass(xl, src, dst, with_count=True)
    hl, hr, scale = _mid_tc(agg1p, cntp.reshape(-1, 1), xr, bl1,
                            Wl2.T, Wr2.T)
    (agg2p,) = _edge_pass(hl, src, dst, with_count=False)
    return _final_tc(agg2p, scale, hr, bl2, batch.reshape(1, n), Wfc.T, bfc)


# speedup vs baseline: 1.5425x; 1.0774x over previous
import jax
import jax.numpy as jnp
from jax import lax
from jax.experimental import pallas as pl
from jax.experimental.pallas import tpu as pltpu
from jax.experimental.pallas import tpu_sc as plsc

_NC = 2
_NS = 16
_G = 64


def _pick_chunk(ept):
    for ch in range(128, 0, -1):
        nch = ept // ch
        if ept % ch == 0 and nch % 4 == 0 and nch >= 8:
            return ch
    raise ValueError(f"no legal chunk size for {ept} edges per tile")


def _edge_pass(feats, edge_index, with_count):
    n, h = feats.shape
    e = edge_index.shape[1]
    nw = _NC * _NS
    assert e % nw == 0 and n % _NS == 0
    ept = e // nw
    ch = _pick_chunk(ept)
    nch = ept // ch
    rp = (n // 8 // _NS) * 8
    rl = n - rp * (_NS - 1)

    mesh = plsc.VectorSubcoreMesh(core_axis_name="c", subcore_axis_name="s",
                                  num_cores=_NC, num_subcores=_NS)

    assert nch % 4 == 0 and nch >= 8
    _NB = 4

    out_type = [jax.ShapeDtypeStruct((_NC * n, h), jnp.float32)]
    scratch = [
        pltpu.VMEM_SHARED((n, h), jnp.float32),
        pltpu.VMEM((nch, ch), jnp.int32),
        pltpu.VMEM((nch, ch), jnp.int32),
        pltpu.VMEM((_NB, ch, h), jnp.float32),
        pltpu.VMEM((rl, h), jnp.float32),
    ]
    scratch += [pltpu.SemaphoreType.DMA] * _NB
    scratch += [pltpu.SemaphoreType.DMA] * _NB
    if with_count:
        out_type.append(jax.ShapeDtypeStruct((_NC * n,), jnp.float32))
        scratch += [
            pltpu.VMEM_SHARED((n,), jnp.float32),
            pltpu.VMEM((ch,), jnp.float32),
            pltpu.VMEM((rl,), jnp.float32),
        ]
        scratch += [pltpu.SemaphoreType.DMA] * _NB

    def body(feats_hbm, zrows_hbm, zcnt_hbm, ones_hbm, edges_hbm, *rest):
        if with_count:
            (agg_out, cnt_out, agg_sh, src_v, dst_v, rows_v, zb_v,
             *sems) = rest
            gsems = sems[:_NB]
            ssems = sems[_NB:2 * _NB]
            cnt_sh, ones_v, zc_v = sems[2 * _NB:2 * _NB + 3]
            csems = sems[2 * _NB + 3:]
        else:
            (agg_out, agg_sh, src_v, dst_v, rows_v, zb_v, *sems) = rest
            gsems = sems[:_NB]
            ssems = sems[_NB:]
            csems = (None,) * _NB
        c = lax.axis_index("c")
        s = lax.axis_index("s")
        wid = c * _NS + s

        pltpu.sync_copy(edges_hbm.at[0, pl.ds(wid * nch, nch)], src_v)
        pltpu.sync_copy(edges_hbm.at[1, pl.ds(wid * nch, nch)], dst_v)

        pltpu.sync_copy(zrows_hbm, zb_v)
        if with_count:
            pltpu.sync_copy(zcnt_hbm, zc_v)
            pltpu.sync_copy(ones_hbm, ones_v)

        @pl.when(s < _NS - 1)
        def _():
            pltpu.sync_copy(zb_v.at[pl.ds(0, rp)],
                            agg_sh.at[pl.ds(s * rp, rp)])
            if with_count:
                pltpu.sync_copy(zc_v.at[pl.ds(0, rp)],
                                cnt_sh.at[pl.ds(s * rp, rp)])

        @pl.when(s == _NS - 1)
        def _():
            pltpu.sync_copy(zb_v, agg_sh.at[pl.ds((_NS - 1) * rp, rl)])
            if with_count:
                pltpu.sync_copy(zc_v, cnt_sh.at[pl.ds((_NS - 1) * rp, rl)])

        plsc.subcore_barrier()

        def fire_gather(j, b):
            pltpu.async_copy(feats_hbm.at[src_v.at[j]],
                             rows_v.at[b], gsems[b])

        def wait_gather(j, b):
            pltpu.make_async_copy(feats_hbm.at[src_v.at[j]],
                                  rows_v.at[b], gsems[b]).wait()

        def fire_scatter(j, b):
            pltpu.async_copy(rows_v.at[b], agg_sh.at[dst_v.at[j]],
                             ssems[b], add=True)
            if with_count:
                pltpu.async_copy(ones_v, cnt_sh.at[dst_v.at[j]],
                                 csems[b], add=True)

        def wait_scatter(j, b):
            pltpu.make_async_copy(rows_v.at[b], agg_sh.at[dst_v.at[j]],
                                  ssems[b]).wait()
            if with_count:
                pltpu.make_async_copy(ones_v, cnt_sh.at[dst_v.at[j]],
                                      csems[b]).wait()

        def step(j, u, fire_next=True, wait_prev=True):
            wait_gather(j, u)
            fire_scatter(j, u)
            if wait_prev:
                wait_scatter(j - 1, (u + 3) % _NB)
            if fire_next:
                fire_gather(j + 3, (u + 3) % _NB)

        for b in range(_NB - 1):
            fire_gather(b, b)
        step(0, 0, wait_prev=False)
        for j in range(1, _NB):
            step(j, j % _NB)

        def quad(g, carry):
            j0 = _NB * g
            for u in range(_NB):
                j = j0 + u

                @pl.when(j + 3 < nch)
                def _():
                    step(j, u)

                @pl.when(j + 3 >= nch)
                def _():
                    step(j, u, fire_next=False)
            return carry

        lax.fori_loop(1, nch // _NB, quad, 0)
        wait_scatter(nch - 1, (_NB - 1) % _NB)
        plsc.subcore_barrier()

        @pl.when(s < _NS - 1)
        def _():
            pltpu.sync_copy(agg_sh.at[pl.ds(s * rp, rp)],
                            zb_v.at[pl.ds(0, rp)])
            pltpu.sync_copy(zb_v.at[pl.ds(0, rp)],
                            agg_out.at[pl.ds(c * n + s * rp, rp)])
            if with_count:
                pltpu.sync_copy(cnt_sh.at[pl.ds(s * rp, rp)],
                                zc_v.at[pl.ds(0, rp)])
                pltpu.sync_copy(zc_v.at[pl.ds(0, rp)],
                                cnt_out.at[pl.ds(c * n + s * rp, rp)])

        @pl.when(s == _NS - 1)
        def _():
            pltpu.sync_copy(agg_sh.at[pl.ds((_NS - 1) * rp, rl)], zb_v)
            pltpu.sync_copy(zb_v,
                            agg_out.at[pl.ds(c * n + (_NS - 1) * rp, rl)])
            if with_count:
                pltpu.sync_copy(cnt_sh.at[pl.ds((_NS - 1) * rp, rl)], zc_v)
                pltpu.sync_copy(
                    zc_v, cnt_out.at[pl.ds(c * n + (_NS - 1) * rp, rl)])

    run = pl.kernel(
        body, out_type=tuple(out_type), mesh=mesh,
        scratch_types=tuple(scratch),
        compiler_params=pltpu.CompilerParams(use_tc_tiling_on_sc=False))
    zrows = jnp.zeros((rl, h), jnp.float32)
    zcnt = jnp.zeros((rl,), jnp.float32)
    ones = jnp.ones((ch,), jnp.float32)
    return run(feats, zrows, zcnt, ones, edge_index.reshape(2, -1, ch))


def _proj_tc(x, wl1t, wr1t):
    n = x.shape[0]
    h = wl1t.shape[1]

    def body(x_ref, wl_ref, wr_ref, xl_ref, xr_ref):
        xv = x_ref[...]
        xl_ref[...] = jnp.dot(xv, wl_ref[...],
                              preferred_element_type=jnp.float32)
        xr_ref[...] = jnp.dot(xv, wr_ref[...],
                              preferred_element_type=jnp.float32)

    return pl.pallas_call(
        body,
        out_shape=(jax.ShapeDtypeStruct((n, h), jnp.float32),
                   jax.ShapeDtypeStruct((n, h), jnp.float32)),
    )(x, wl1t, wr1t)


def _mid_tc(agg1p, cnt2, xr, bl1, wl2t, wr2t):
    n, h = xr.shape

    def body(agg_ref, cnt_ref, xr_ref, b_ref, wl_ref, wr_ref,
             hl_ref, hr_ref, sc_ref):
        a = agg_ref[...]
        agg = a[:n] + a[n:]
        cv = cnt_ref[...]
        cnt = jnp.maximum(cv[:n] + cv[n:], 1.0)
        inv = 1.0 / cnt
        hh = jnp.maximum(agg * inv + b_ref[...] + xr_ref[...], 0.0)
        hl_ref[...] = jnp.dot(hh, wl_ref[...],
                              preferred_element_type=jnp.float32)
        hr_ref[...] = jnp.dot(hh, wr_ref[...],
                              preferred_element_type=jnp.float32)
        sc_ref[...] = inv

    return pl.pallas_call(
        body,
        out_shape=(jax.ShapeDtypeStruct((n, h), jnp.float32),
                   jax.ShapeDtypeStruct((n, h), jnp.float32),
                   jax.ShapeDtypeStruct((n, 1), jnp.float32)),
    )(agg1p, cnt2, xr, bl1.reshape(1, h), wl2t, wr2t)


def _final_tc(agg2p, scale, hr, bl2, batch_row, wfct, bfc):
    n, h = hr.shape
    co = wfct.shape[1]

    def body(agg_ref, sc_ref, hr_ref, b_ref, bt_ref, wf_ref, bf_ref, o_ref):
        a = agg_ref[...]
        h2 = (a[:n] + a[n:]) * sc_ref[...] + b_ref[...] + hr_ref[...]
        ids = bt_ref[...]
        iot = lax.broadcasted_iota(jnp.int32, (_G, n), 0)
        oh = jnp.where(iot == ids, 1.0, 0.0)
        pooled = jnp.dot(oh, h2, preferred_element_type=jnp.float32)
        gcnt = jnp.sum(oh, axis=1, keepdims=True)
        pooled = pooled / jnp.maximum(gcnt, 1.0)
        logits = jnp.dot(pooled, wf_ref[...],
                         preferred_element_type=jnp.float32) + bf_ref[...]
        m = jnp.max(logits, axis=1, keepdims=True)
        sh = logits - m
        o_ref[...] = sh - jnp.log(jnp.sum(jnp.exp(sh), axis=1, keepdims=True))

    return pl.pallas_call(
        body,
        out_shape=jax.ShapeDtypeStruct((_G, co), jnp.float32),
    )(agg2p, scale, hr, bl2.reshape(1, h), batch_row, wfct,
      bfc.reshape(1, co))


def kernel(x, edge_index, batch, Wl1, bl1, Wr1, Wl2, bl2, Wr2, Wfc, bfc):
    n, _ = x.shape
    h = Wl1.shape[0]

    xl, xr = _proj_tc(x, Wl1.T, Wr1.T)
    agg1p, cntp = _edge_pass(xl, edge_index, with_count=True)
    hl, hr, scale = _mid_tc(agg1p, cntp.reshape(-1, 1), xr, bl1,
                            Wl2.T, Wr2.T)
    (agg2p,) = _edge_pass(hl, edge_index, with_count=False)
    return _final_tc(agg2p, scale, hr, bl2, batch.reshape(1, n), Wfc.T, bfc)


# speedup vs baseline: 2.2470x; 1.4568x over previous
import jax
import jax.numpy as jnp
from jax import lax
from jax.experimental import pallas as pl
from jax.experimental.pallas import tpu as pltpu
from jax.experimental.pallas import tpu_sc as plsc

_NC = 2
_NS = 16
_G = 64


def _pick_chunk(ept):
    for ch in range(128, 0, -1):
        nch = ept // ch
        if ept % ch == 0 and nch % 4 == 0 and nch >= 8:
            return ch
    raise ValueError(f"no legal chunk size for {ept} edges per tile")


def _edge_pass(feats, edge_index, with_count):
    n, h = feats.shape
    e = edge_index.shape[1]
    nw = _NC * _NS
    assert e % nw == 0 and n % _NS == 0
    ept = e // nw
    ch = _pick_chunk(ept)
    nch = ept // ch
    rp = (n // 8 // _NS) * 8
    rl = n - rp * (_NS - 1)

    mesh = plsc.VectorSubcoreMesh(core_axis_name="c", subcore_axis_name="s",
                                  num_cores=_NC, num_subcores=_NS)

    assert nch % 4 == 0 and nch >= 8
    _NB = 4

    out_type = [jax.ShapeDtypeStruct((_NC * n, h), jnp.float32)]
    scratch = [
        pltpu.VMEM_SHARED((n, h), jnp.float32),
        pltpu.VMEM((nch, ch), jnp.int32),
        pltpu.VMEM((nch, ch), jnp.int32),
        pltpu.VMEM((_NB, ch, h), jnp.float32),
        pltpu.VMEM((rl, h), jnp.float32),
    ]
    scratch += [pltpu.SemaphoreType.DMA] * _NB
    scratch += [pltpu.SemaphoreType.DMA] * _NB
    if with_count:
        out_type.append(jax.ShapeDtypeStruct((_NC * n,), jnp.float32))
        scratch += [
            pltpu.VMEM_SHARED((n,), jnp.float32),
            pltpu.VMEM((ch,), jnp.float32),
            pltpu.VMEM((rl,), jnp.float32),
        ]
        scratch += [pltpu.SemaphoreType.DMA] * _NB

    def body(feats_hbm, zrows_hbm, zcnt_hbm, ones_hbm, edges_hbm, *rest):
        if with_count:
            (agg_out, cnt_out, agg_sh, src_v, dst_v, rows_v, zb_v,
             *sems) = rest
            gsems = sems[:_NB]
            ssems = sems[_NB:2 * _NB]
            cnt_sh, ones_v, zc_v = sems[2 * _NB:2 * _NB + 3]
            csems = sems[2 * _NB + 3:]
        else:
            (agg_out, agg_sh, src_v, dst_v, rows_v, zb_v, *sems) = rest
            gsems = sems[:_NB]
            ssems = sems[_NB:]
            csems = (None,) * _NB
        c = lax.axis_index("c")
        s = lax.axis_index("s")
        wid = c * _NS + s

        pltpu.sync_copy(edges_hbm.at[0, pl.ds(wid * nch, nch)], src_v)
        pltpu.sync_copy(edges_hbm.at[1, pl.ds(wid * nch, nch)], dst_v)

        pltpu.sync_copy(zrows_hbm, zb_v)
        if with_count:
            pltpu.sync_copy(zcnt_hbm, zc_v)
            pltpu.sync_copy(ones_hbm, ones_v)

        @pl.when(s < _NS - 1)
        def _():
            pltpu.sync_copy(zb_v.at[pl.ds(0, rp)],
                            agg_sh.at[pl.ds(s * rp, rp)])
            if with_count:
                pltpu.sync_copy(zc_v.at[pl.ds(0, rp)],
                                cnt_sh.at[pl.ds(s * rp, rp)])

        @pl.when(s == _NS - 1)
        def _():
            pltpu.sync_copy(zb_v, agg_sh.at[pl.ds((_NS - 1) * rp, rl)])
            if with_count:
                pltpu.sync_copy(zc_v, cnt_sh.at[pl.ds((_NS - 1) * rp, rl)])

        plsc.subcore_barrier()

        def fire_gather(j, b):
            pltpu.async_copy(feats_hbm.at[src_v.at[j]],
                             rows_v.at[b], gsems[b])

        def wait_gather(j, b):
            pltpu.make_async_copy(feats_hbm.at[src_v.at[j]],
                                  rows_v.at[b], gsems[b]).wait()

        def fire_scatter(j, b):
            pltpu.async_copy(rows_v.at[b], agg_sh.at[dst_v.at[j]],
                             ssems[b], add=True)
            if with_count:
                pltpu.async_copy(ones_v, cnt_sh.at[dst_v.at[j]],
                                 csems[b], add=True)

        def wait_scatter(j, b):
            pltpu.make_async_copy(rows_v.at[b], agg_sh.at[dst_v.at[j]],
                                  ssems[b]).wait()
            if with_count:
                pltpu.make_async_copy(ones_v, cnt_sh.at[dst_v.at[j]],
                                      csems[b]).wait()

        def step(j, u, fire_next=True, wait_prev=True):
            wait_gather(j, u)
            fire_scatter(j, u)
            if wait_prev:
                wait_scatter(j - 1, (u + 3) % _NB)
            if fire_next:
                fire_gather(j + 3, (u + 3) % _NB)

        for b in range(_NB - 1):
            fire_gather(b, b)
        step(0, 0, wait_prev=False)
        for j in range(1, _NB):
            step(j, j % _NB)

        def quad(g, carry):
            j0 = _NB * g
            for u in range(_NB):
                j = j0 + u

                @pl.when(j + 3 < nch)
                def _():
                    step(j, u)

                @pl.when(j + 3 >= nch)
                def _():
                    step(j, u, fire_next=False)
            return carry

        lax.fori_loop(1, nch // _NB, quad, 0)
        wait_scatter(nch - 1, (_NB - 1) % _NB)
        plsc.subcore_barrier()

        @pl.when(s < _NS - 1)
        def _():
            pltpu.sync_copy(agg_sh.at[pl.ds(s * rp, rp)],
                            zb_v.at[pl.ds(0, rp)])
            pltpu.sync_copy(zb_v.at[pl.ds(0, rp)],
                            agg_out.at[pl.ds(c * n + s * rp, rp)])
            if with_count:
                pltpu.sync_copy(cnt_sh.at[pl.ds(s * rp, rp)],
                                zc_v.at[pl.ds(0, rp)])
                pltpu.sync_copy(zc_v.at[pl.ds(0, rp)],
                                cnt_out.at[pl.ds(c * n + s * rp, rp)])

        @pl.when(s == _NS - 1)
        def _():
            pltpu.sync_copy(agg_sh.at[pl.ds((_NS - 1) * rp, rl)], zb_v)
            pltpu.sync_copy(zb_v,
                            agg_out.at[pl.ds(c * n + (_NS - 1) * rp, rl)])
            if with_count:
                pltpu.sync_copy(cnt_sh.at[pl.ds((_NS - 1) * rp, rl)], zc_v)
                pltpu.sync_copy(
                    zc_v, cnt_out.at[pl.ds(c * n + (_NS - 1) * rp, rl)])

    run = pl.kernel(
        body, out_type=tuple(out_type), mesh=mesh,
        scratch_types=tuple(scratch),
        compiler_params=pltpu.CompilerParams(use_tc_tiling_on_sc=False))
    zrows = jnp.zeros((rl, h), jnp.float32)
    zcnt = jnp.zeros((rl,), jnp.float32)
    ones = jnp.ones((ch,), jnp.float32)
    return run(feats, zrows, zcnt, ones, edge_index.reshape(2, -1, ch))


def _proj_tc(x, m1l, m1r):
    n = x.shape[0]
    p = n // 8

    def body(x_ref, wl_ref, wr_ref, xl_ref, xr_ref):
        xf = x_ref[...].reshape(p, 8 * x_ref.shape[1])
        xl_ref[...] = jnp.dot(xf, wl_ref[...],
                              preferred_element_type=jnp.float32)
        xr_ref[...] = jnp.dot(xf, wr_ref[...],
                              preferred_element_type=jnp.float32)

    return pl.pallas_call(
        body,
        out_shape=(jax.ShapeDtypeStruct((p, 128), jnp.float32),
                   jax.ShapeDtypeStruct((p, 128), jnp.float32)),
    )(x, m1l, m1r)


def _mid_tc(agg1p, cnt8, xr_p, bl1_8, m2l, m2r, rmat):
    p = xr_p.shape[0]

    def body(agg_ref, cnt_ref, xr_ref, b_ref, wl_ref, wr_ref, r_ref,
             hl_ref, hr_ref, sc_ref):
        a = agg_ref[...]
        agg = a[:p] + a[p:]
        cv = cnt_ref[...]
        inv8 = 1.0 / jnp.maximum(cv[:p] + cv[p:], 1.0)
        inv = jnp.dot(inv8, r_ref[...], preferred_element_type=jnp.float32)
        hh = jnp.maximum(agg * inv + b_ref[...] + xr_ref[...], 0.0)
        hl_ref[...] = jnp.dot(hh, wl_ref[...],
                              preferred_element_type=jnp.float32)
        hr_ref[...] = jnp.dot(hh, wr_ref[...],
                              preferred_element_type=jnp.float32)
        sc_ref[...] = inv

    return pl.pallas_call(
        body,
        out_shape=(jax.ShapeDtypeStruct((p, 128), jnp.float32),
                   jax.ShapeDtypeStruct((p, 128), jnp.float32),
                   jax.ShapeDtypeStruct((p, 128), jnp.float32)),
    )(agg1p, cnt8, xr_p, bl1_8, m2l, m2r, rmat)


def _final_tc(agg2p, inv_p, hr_p, bl2_8, batch8t, wfct, bfc):
    p = hr_p.shape[0]
    h = 16
    co = wfct.shape[1]

    def body(agg_ref, sc_ref, hr_ref, b_ref, bt_ref, wf_ref, bf_ref, o_ref):
        a = agg_ref[...]
        h2 = (a[:p] + a[p:]) * sc_ref[...] + b_ref[...] + hr_ref[...]
        bt = bt_ref[...]
        iot = lax.broadcasted_iota(jnp.int32, (_G, p), 0)
        pooled = jnp.zeros((_G, h), jnp.float32)
        gcnt = jnp.zeros((_G, 1), jnp.float32)
        for sub in range(8):
            oh = jnp.where(iot == bt[sub][None, :], 1.0, 0.0)
            pa = jnp.dot(oh, h2, preferred_element_type=jnp.float32)
            pooled = pooled + pa[:, h * sub:h * (sub + 1)]
            gcnt = gcnt + jnp.sum(oh, axis=1, keepdims=True)
        pooled = pooled / jnp.maximum(gcnt, 1.0)
        logits = jnp.dot(pooled, wf_ref[...],
                         preferred_element_type=jnp.float32) + bf_ref[...]
        m = jnp.max(logits, axis=1, keepdims=True)
        sh = logits - m
        o_ref[...] = sh - jnp.log(jnp.sum(jnp.exp(sh), axis=1, keepdims=True))

    return pl.pallas_call(
        body,
        out_shape=jax.ShapeDtypeStruct((_G, co), jnp.float32),
    )(agg2p, inv_p, hr_p, bl2_8, batch8t, wfct, bfc.reshape(1, co))


def kernel(x, edge_index, batch, Wl1, bl1, Wr1, Wl2, bl2, Wr2, Wfc, bfc):
    n, _ = x.shape
    h = Wl1.shape[0]
    p = n // 8
    eye8 = jnp.eye(8, dtype=jnp.float32)
    m1l = jnp.kron(eye8, Wl1.T)
    m1r = jnp.kron(eye8, Wr1.T)
    m2l = jnp.kron(eye8, Wl2.T)
    m2r = jnp.kron(eye8, Wr2.T)
    rmat = jnp.kron(eye8, jnp.ones((1, h), jnp.float32))
    bl1_8 = jnp.tile(bl1, 8).reshape(1, 128)
    bl2_8 = jnp.tile(bl2, 8).reshape(1, 128)

    xl_p, xr_p = _proj_tc(x, m1l, m1r)
    agg1p, cntp = _edge_pass(xl_p.reshape(n, h), edge_index, with_count=True)
    hl_p, hr_p, inv_p = _mid_tc(agg1p.reshape(2 * p, 128),
                                cntp.reshape(2 * p, 8), xr_p, bl1_8,
                                m2l, m2r, rmat)
    (agg2p,) = _edge_pass(hl_p.reshape(n, h), edge_index, with_count=False)
    return _final_tc(agg2p.reshape(2 * p, 128), inv_p, hr_p, bl2_8,
                     batch.reshape(p, 8).T, Wfc.T, bfc)


# speedup vs baseline: 2.7566x; 1.2268x over previous
import jax
import jax.numpy as jnp
from jax import lax
from jax.experimental import pallas as pl
from jax.experimental.pallas import tpu as pltpu
from jax.experimental.pallas import tpu_sc as plsc

_NC = 2
_NS = 16
_G = 64


def _pick_chunk(ept):
    for ch in range(128, 0, -1):
        nch = ept // ch
        if ept % ch == 0 and nch % 4 == 0 and nch >= 8:
            return ch
    raise ValueError(f"no legal chunk size for {ept} edges per tile")


def _edge_pass(feats, edge_index, with_count):
    n, h = feats.shape
    e = edge_index.shape[1]
    nw = _NC * _NS
    assert e % nw == 0 and n % _NS == 0
    ept = e // nw
    ch = _pick_chunk(ept)
    nch = ept // ch
    rp = (n // 8 // _NS) * 8
    rl = n - rp * (_NS - 1)

    mesh = plsc.VectorSubcoreMesh(core_axis_name="c", subcore_axis_name="s",
                                  num_cores=_NC, num_subcores=_NS)

    _NB = 8
    assert nch % _NB == 0 and nch >= 2 * _NB

    out_type = [jax.ShapeDtypeStruct((_NC * n, h), jnp.float32)]
    scratch = [
        pltpu.VMEM_SHARED((n, h), jnp.float32),
        pltpu.VMEM((nch, ch), jnp.int32),
        pltpu.VMEM((nch, ch), jnp.int32),
        pltpu.VMEM((_NB, ch, h), jnp.float32),
        pltpu.VMEM((rl, h), jnp.float32),
    ]
    scratch += [pltpu.SemaphoreType.DMA] * _NB
    scratch += [pltpu.SemaphoreType.DMA] * _NB
    if with_count:
        out_type.append(jax.ShapeDtypeStruct((_NC * n,), jnp.float32))
        scratch += [
            pltpu.VMEM_SHARED((n,), jnp.float32),
            pltpu.VMEM((ch,), jnp.float32),
            pltpu.VMEM((rl,), jnp.float32),
        ]
        scratch += [pltpu.SemaphoreType.DMA] * _NB

    def body(feats_hbm, zrows_hbm, zcnt_hbm, ones_hbm, edges_hbm, *rest):
        if with_count:
            (agg_out, cnt_out, agg_sh, src_v, dst_v, rows_v, zb_v,
             *sems) = rest
            gsems = sems[:_NB]
            ssems = sems[_NB:2 * _NB]
            cnt_sh, ones_v, zc_v = sems[2 * _NB:2 * _NB + 3]
            csems = sems[2 * _NB + 3:]
        else:
            (agg_out, agg_sh, src_v, dst_v, rows_v, zb_v, *sems) = rest
            gsems = sems[:_NB]
            ssems = sems[_NB:]
            csems = (None,) * _NB
        c = lax.axis_index("c")
        s = lax.axis_index("s")
        wid = c * _NS + s

        pltpu.sync_copy(edges_hbm.at[0, pl.ds(wid * nch, nch)], src_v)
        pltpu.sync_copy(edges_hbm.at[1, pl.ds(wid * nch, nch)], dst_v)

        pltpu.sync_copy(zrows_hbm, zb_v)
        if with_count:
            pltpu.sync_copy(zcnt_hbm, zc_v)
            pltpu.sync_copy(ones_hbm, ones_v)

        @pl.when(s < _NS - 1)
        def _():
            pltpu.sync_copy(zb_v.at[pl.ds(0, rp)],
                            agg_sh.at[pl.ds(s * rp, rp)])
            if with_count:
                pltpu.sync_copy(zc_v.at[pl.ds(0, rp)],
                                cnt_sh.at[pl.ds(s * rp, rp)])

        @pl.when(s == _NS - 1)
        def _():
            pltpu.sync_copy(zb_v, agg_sh.at[pl.ds((_NS - 1) * rp, rl)])
            if with_count:
                pltpu.sync_copy(zc_v, cnt_sh.at[pl.ds((_NS - 1) * rp, rl)])

        plsc.subcore_barrier()

        def fire_gather(j, b):
            pltpu.async_copy(feats_hbm.at[src_v.at[j]],
                             rows_v.at[b], gsems[b])

        def wait_gather(j, b):
            pltpu.make_async_copy(feats_hbm.at[src_v.at[j]],
                                  rows_v.at[b], gsems[b]).wait()

        def fire_scatter(j, b):
            pltpu.async_copy(rows_v.at[b], agg_sh.at[dst_v.at[j]],
                             ssems[b], add=True)
            if with_count:
                pltpu.async_copy(ones_v, cnt_sh.at[dst_v.at[j]],
                                 csems[b], add=True)

        def wait_scatter(j, b):
            pltpu.make_async_copy(rows_v.at[b], agg_sh.at[dst_v.at[j]],
                                  ssems[b]).wait()
            if with_count:
                pltpu.make_async_copy(ones_v, cnt_sh.at[dst_v.at[j]],
                                      csems[b]).wait()

        def step(j, u, fire_next=True, wait_prev=True):
            wait_gather(j, u)
            fire_scatter(j, u)
            if wait_prev:
                wait_scatter(j - 1, (u + _NB - 1) % _NB)
            if fire_next:
                fire_gather(j + _NB - 1, (u + _NB - 1) % _NB)

        for b in range(_NB - 1):
            fire_gather(b, b)
        step(0, 0, wait_prev=False)
        for j in range(1, _NB):
            step(j, j % _NB)

        def quad(g, carry):
            j0 = _NB * g
            for u in range(_NB):
                j = j0 + u

                @pl.when(j + _NB - 1 < nch)
                def _():
                    step(j, u)

                @pl.when(j + _NB - 1 >= nch)
                def _():
                    step(j, u, fire_next=False)
            return carry

        lax.fori_loop(1, nch // _NB, quad, 0)
        wait_scatter(nch - 1, (_NB - 1) % _NB)
        plsc.subcore_barrier()

        @pl.when(s < _NS - 1)
        def _():
            pltpu.sync_copy(agg_sh.at[pl.ds(s * rp, rp)],
                            zb_v.at[pl.ds(0, rp)])
            pltpu.sync_copy(zb_v.at[pl.ds(0, rp)],
                            agg_out.at[pl.ds(c * n + s * rp, rp)])
            if with_count:
                pltpu.sync_copy(cnt_sh.at[pl.ds(s * rp, rp)],
                                zc_v.at[pl.ds(0, rp)])
                pltpu.sync_copy(zc_v.at[pl.ds(0, rp)],
                                cnt_out.at[pl.ds(c * n + s * rp, rp)])

        @pl.when(s == _NS - 1)
        def _():
            pltpu.sync_copy(agg_sh.at[pl.ds((_NS - 1) * rp, rl)], zb_v)
            pltpu.sync_copy(zb_v,
                            agg_out.at[pl.ds(c * n + (_NS - 1) * rp, rl)])
            if with_count:
                pltpu.sync_copy(cnt_sh.at[pl.ds((_NS - 1) * rp, rl)], zc_v)
                pltpu.sync_copy(
                    zc_v, cnt_out.at[pl.ds(c * n + (_NS - 1) * rp, rl)])

    run = pl.kernel(
        body, out_type=tuple(out_type), mesh=mesh,
        scratch_types=tuple(scratch),
        compiler_params=pltpu.CompilerParams(use_tc_tiling_on_sc=False))
    zrows = jnp.zeros((rl, h), jnp.float32)
    zcnt = jnp.zeros((rl,), jnp.float32)
    ones = jnp.ones((ch,), jnp.float32)
    return run(feats, zrows, zcnt, ones, edge_index.reshape(2, -1, ch))


def _proj_tc(x, m1l, m1r):
    n = x.shape[0]
    p = n // 8

    def body(x_ref, wl_ref, wr_ref, xl_ref, xr_ref):
        xf = x_ref[...].reshape(p, 8 * x_ref.shape[1])
        xl_ref[...] = jnp.dot(xf, wl_ref[...],
                              preferred_element_type=jnp.float32)
        xr_ref[...] = jnp.dot(xf, wr_ref[...],
                              preferred_element_type=jnp.float32)

    return pl.pallas_call(
        body,
        out_shape=(jax.ShapeDtypeStruct((p, 128), jnp.float32),
                   jax.ShapeDtypeStruct((p, 128), jnp.float32)),
    )(x, m1l, m1r)


def _mid_tc(agg1p, cnt8, xr_p, bl1_8, m2l, m2r, rmat):
    p = xr_p.shape[0]

    def body(agg_ref, cnt_ref, xr_ref, b_ref, wl_ref, wr_ref, r_ref,
             hl_ref, hr_ref, sc_ref):
        a = agg_ref[...]
        agg = a[:p] + a[p:]
        cv = cnt_ref[...]
        inv8 = 1.0 / jnp.maximum(cv[:p] + cv[p:], 1.0)
        inv = jnp.dot(inv8, r_ref[...], preferred_element_type=jnp.float32)
        hh = jnp.maximum(agg * inv + b_ref[...] + xr_ref[...], 0.0)
        hl_ref[...] = jnp.dot(hh, wl_ref[...],
                              preferred_element_type=jnp.float32)
        hr_ref[...] = jnp.dot(hh, wr_ref[...],
                              preferred_element_type=jnp.float32)
        sc_ref[...] = inv

    return pl.pallas_call(
        body,
        out_shape=(jax.ShapeDtypeStruct((p, 128), jnp.float32),
                   jax.ShapeDtypeStruct((p, 128), jnp.float32),
                   jax.ShapeDtypeStruct((p, 128), jnp.float32)),
    )(agg1p, cnt8, xr_p, bl1_8, m2l, m2r, rmat)


def _final_tc(agg2p, inv_p, hr_p, bl2_8, batch8t, wfct, bfc):
    p = hr_p.shape[0]
    h = 16
    co = wfct.shape[1]

    def body(agg_ref, sc_ref, hr_ref, b_ref, bt_ref, wf_ref, bf_ref, o_ref):
        a = agg_ref[...]
        h2 = (a[:p] + a[p:]) * sc_ref[...] + b_ref[...] + hr_ref[...]
        bt = bt_ref[...]
        iot = lax.broadcasted_iota(jnp.int32, (_G, p), 0)
        pooled = jnp.zeros((_G, h), jnp.float32)
        gcnt = jnp.zeros((_G, 1), jnp.float32)
        for sub in range(8):
            oh = jnp.where(iot == bt[sub][None, :], 1.0, 0.0)
            pa = jnp.dot(oh, h2, preferred_element_type=jnp.float32)
            pooled = pooled + pa[:, h * sub:h * (sub + 1)]
            gcnt = gcnt + jnp.sum(oh, axis=1, keepdims=True)
        pooled = pooled / jnp.maximum(gcnt, 1.0)
        logits = jnp.dot(pooled, wf_ref[...],
                         preferred_element_type=jnp.float32) + bf_ref[...]
        m = jnp.max(logits, axis=1, keepdims=True)
        sh = logits - m
        o_ref[...] = sh - jnp.log(jnp.sum(jnp.exp(sh), axis=1, keepdims=True))

    return pl.pallas_call(
        body,
        out_shape=jax.ShapeDtypeStruct((_G, co), jnp.float32),
    )(agg2p, inv_p, hr_p, bl2_8, batch8t, wfct, bfc.reshape(1, co))


def kernel(x, edge_index, batch, Wl1, bl1, Wr1, Wl2, bl2, Wr2, Wfc, bfc):
    n, _ = x.shape
    h = Wl1.shape[0]
    p = n // 8
    eye8 = jnp.eye(8, dtype=jnp.float32)
    m1l = jnp.kron(eye8, Wl1.T)
    m1r = jnp.kron(eye8, Wr1.T)
    m2l = jnp.kron(eye8, Wl2.T)
    m2r = jnp.kron(eye8, Wr2.T)
    rmat = jnp.kron(eye8, jnp.ones((1, h), jnp.float32))
    bl1_8 = jnp.tile(bl1, 8).reshape(1, 128)
    bl2_8 = jnp.tile(bl2, 8).reshape(1, 128)

    xl_p, xr_p = _proj_tc(x, m1l, m1r)
    agg1p, cntp = _edge_pass(xl_p.reshape(n, h), edge_index, with_count=True)
    hl_p, hr_p, inv_p = _mid_tc(agg1p.reshape(2 * p, 128),
                                cntp.reshape(2 * p, 8), xr_p, bl1_8,
                                m2l, m2r, rmat)
    (agg2p,) = _edge_pass(hl_p.reshape(n, h), edge_index, with_count=False)
    return _final_tc(agg2p.reshape(2 * p, 128), inv_p, hr_p, bl2_8,
                     batch.reshape(p, 8).T, Wfc.T, bfc)
